# baseline (device time: 18685 ns/iter reference)
import jax
import jax.numpy as jnp
from jax import lax
from jax.experimental import pallas as pl
from jax.experimental.pallas import tpu as pltpu

N_DEV = 4
B = 2
SQ = 256
SKV = 256
HALO = 128
SFULL = SKV + 2 * HALO
HQ = 4
DH = 64
HD = HQ * DH
DM = 512
WINDOW = 128
SKV_GLOBAL = N_DEV * SKV


def kernel(x, Wq, K_ext, V_ext, Wo):
    K2 = K_ext.reshape(B, SKV, HD)
    V2 = V_ext.reshape(B, SKV, HD)

    def body(x_ref, wq_ref, k_ref, v_ref, wo_ref, out_ref,
             k_full, v_full, send_sems, recv_sems):
        my = lax.axis_index("i")
        left = (my + N_DEV - 1) % N_DEV
        right = (my + 1) % N_DEV

        barrier_sem = pltpu.get_barrier_semaphore()
        for nbr in (left, right):
            pl.semaphore_signal(
                barrier_sem, inc=1,
                device_id=(nbr,), device_id_type=pl.DeviceIdType.MESH,
            )
        pl.semaphore_wait(barrier_sem, 2)

        k_full[:, HALO:HALO + SKV, :] = k_ref[...].astype(jnp.bfloat16)
        v_full[:, HALO:HALO + SKV, :] = v_ref[...].astype(jnp.bfloat16)

        def halo_rdma(buf, src_off, dst_off, dev, sem_i):
            return pltpu.make_async_remote_copy(
                src_ref=buf.at[:, pl.ds(src_off, HALO), :],
                dst_ref=buf.at[:, pl.ds(dst_off, HALO), :],
                send_sem=send_sems.at[sem_i],
                recv_sem=recv_sems.at[sem_i],
                device_id=(dev,),
                device_id_type=pl.DeviceIdType.MESH,
            )

        rdmas = [
            halo_rdma(k_full, HALO, SKV + HALO, left, 0),
            halo_rdma(k_full, SKV, 0, right, 1),
            halo_rdma(v_full, HALO, SKV + HALO, left, 2),
            halo_rdma(v_full, SKV, 0, right, 3),
        ]
        for r in rdmas:
            r.start()

        wq = wq_ref[...].astype(jnp.bfloat16)
        wo = wo_ref[...].astype(jnp.bfloat16)
        qs = []
        for b in range(B):
            xb = x_ref[b].astype(jnp.bfloat16)
            qs.append(lax.dot(xb, wq,
                              preferred_element_type=jnp.float32))

        HQH = SQ // 2
        KVW = SKV + HALO
        qi = lax.broadcasted_iota(jnp.int32, (HQH, KVW), 0)
        kj = lax.broadcasted_iota(jnp.int32, (HQH, KVW), 1)
        window = jnp.abs(qi - kj + HALO) <= WINDOW

        half_rdmas = [(rdmas[1], rdmas[3]), (rdmas[0], rdmas[2])]
        for half in range(2):
            for r in half_rdmas[half]:
                r.wait()
            j0 = half * HQH
            kg = kj + j0 + my * SKV - HALO
            mask = window & (kg >= 0) & (kg < SKV_GLOBAL)
            for b in range(B):
                kb = k_full[b, j0:j0 + KVW, :]
                vb = v_full[b, j0:j0 + KVW, :]
                ctx_heads = []
                for h in range(HQ):
                    qh = qs[b][j0:j0 + HQH,
                               h * DH:(h + 1) * DH].astype(jnp.bfloat16)
                    kh = kb[:, h * DH:(h + 1) * DH]
                    s = lax.dot_general(
                        qh, kh, (((1,), (1,)), ((), ())),
                        preferred_element_type=jnp.float32) * 0.125
                    s = jnp.where(mask, s, -1e9)
                    m = jnp.max(s, axis=-1, keepdims=True)
                    w = jnp.exp(s - m)
                    w = w / jnp.sum(w, axis=-1, keepdims=True)
                    ctx_heads.append(lax.dot(
                        w.astype(jnp.bfloat16), vb[:, h * DH:(h + 1) * DH],
                        preferred_element_type=jnp.float32))
                ctx = jnp.concatenate(ctx_heads, axis=1)
                out_ref[b, j0:j0 + HQH, :] = lax.dot(
                    ctx.astype(jnp.bfloat16), wo,
                    preferred_element_type=jnp.float32)

    return pl.pallas_call(
        body,
        out_shape=jax.ShapeDtypeStruct((B, SQ, DM), jnp.float32),
        in_specs=[pl.BlockSpec(memory_space=pltpu.VMEM)] * 5,
        out_specs=pl.BlockSpec(memory_space=pltpu.VMEM),
        scratch_shapes=[
            pltpu.VMEM((B, SFULL, HD), jnp.bfloat16),
            pltpu.VMEM((B, SFULL, HD), jnp.bfloat16),
            pltpu.SemaphoreType.DMA((4,)),
            pltpu.SemaphoreType.DMA((4,)),
        ],
        compiler_params=pltpu.CompilerParams(collective_id=0),
    )(x, Wq, K2, V2, Wo)


# device time: 13909 ns/iter; 1.3434x vs baseline; 1.3434x over previous
import jax
import jax.numpy as jnp
from jax import lax
from jax.experimental import pallas as pl
from jax.experimental.pallas import tpu as pltpu

N_DEV = 4
B = 2
SQ = 256
SKV = 256
HALO = 128
SFULL = SKV + 2 * HALO
HQ = 4
DH = 64
HD = HQ * DH
DM = 512
WINDOW = 128
SKV_GLOBAL = N_DEV * SKV


def kernel(x, Wq, K_ext, V_ext, Wo):
    K2 = K_ext.reshape(B, SKV, HD)
    V2 = V_ext.reshape(B, SKV, HD)

    def body(x_ref, wq_ref, k_ref, v_ref, wo_ref, out_ref,
             k_full, v_full, send_sems, recv_sems):
        my = lax.axis_index("i")
        left = (my + N_DEV - 1) % N_DEV
        right = (my + 1) % N_DEV

        barrier_sem = pltpu.get_barrier_semaphore()
        for nbr in (left, right):
            pl.semaphore_signal(
                barrier_sem, inc=1,
                device_id=(nbr,), device_id_type=pl.DeviceIdType.MESH,
            )
        pl.semaphore_wait(barrier_sem, 2)

        k_full[:, HALO:HALO + SKV, :] = k_ref[...].astype(jnp.bfloat16)
        v_full[:, HALO:HALO + SKV, :] = v_ref[...].astype(jnp.bfloat16)

        def halo_rdma(buf, src_off, dst_off, dev, sem_i):
            return pltpu.make_async_remote_copy(
                src_ref=buf.at[:, pl.ds(src_off, HALO), :],
                dst_ref=buf.at[:, pl.ds(dst_off, HALO), :],
                send_sem=send_sems.at[sem_i],
                recv_sem=recv_sems.at[sem_i],
                device_id=(dev,),
                device_id_type=pl.DeviceIdType.MESH,
            )

        rdmas = [
            halo_rdma(k_full, HALO, SKV + HALO, left, 0),
            halo_rdma(k_full, SKV, 0, right, 1),
            halo_rdma(v_full, HALO, SKV + HALO, left, 2),
            halo_rdma(v_full, SKV, 0, right, 3),
        ]
        for r in rdmas:
            r.start()

        wq = wq_ref[...].astype(jnp.bfloat16)
        wo = wo_ref[...].astype(jnp.bfloat16)
        x2 = x_ref[...].reshape(B * SQ, DM).astype(jnp.bfloat16)
        q_all = (lax.dot(x2, wq, preferred_element_type=jnp.float32)
                 * 0.125).astype(jnp.bfloat16)

        qi = lax.broadcasted_iota(jnp.int32, (SQ, SFULL), 0)
        kj = lax.broadcasted_iota(jnp.int32, (SQ, SFULL), 1)
        window = jnp.abs(qi - kj + HALO) <= WINDOW
        kg = kj + my * SKV - HALO
        mask = window & (kg >= 0) & (kg < SKV_GLOBAL)
        mbias = jnp.where(mask, 0.0, -1e9).astype(jnp.float32)

        for r in rdmas:
            r.wait()

        ctxs = []
        for b in range(B):
            kb = k_full[b]
            vb = v_full[b]
            for h in range(HQ):
                qh = q_all[b * SQ:(b + 1) * SQ, h * DH:(h + 1) * DH]
                kh = kb[:, h * DH:(h + 1) * DH]
                s = lax.dot_general(
                    qh, kh, (((1,), (1,)), ((), ())),
                    preferred_element_type=jnp.float32) + mbias
                w = jnp.exp(s)
                denom = jnp.sum(w, axis=-1, keepdims=True)
                ctx_h = lax.dot(
                    w.astype(jnp.bfloat16), vb[:, h * DH:(h + 1) * DH],
                    preferred_element_type=jnp.float32)
                ctxs.append(ctx_h / denom)
        ctx = jnp.concatenate(
            [jnp.concatenate(ctxs[b * HQ:(b + 1) * HQ], axis=1)
             for b in range(B)], axis=0)
        out = lax.dot(ctx.astype(jnp.bfloat16), wo,
                      preferred_element_type=jnp.float32)
        for b in range(B):
            out_ref[b] = out[b * SQ:(b + 1) * SQ, :]

    return pl.pallas_call(
        body,
        out_shape=jax.ShapeDtypeStruct((B, SQ, DM), jnp.float32),
        in_specs=[pl.BlockSpec(memory_space=pltpu.VMEM)] * 5,
        out_specs=pl.BlockSpec(memory_space=pltpu.VMEM),
        scratch_shapes=[
            pltpu.VMEM((B, SFULL, HD), jnp.bfloat16),
            pltpu.VMEM((B, SFULL, HD), jnp.bfloat16),
            pltpu.SemaphoreType.DMA((4,)),
            pltpu.SemaphoreType.DMA((4,)),
        ],
        compiler_params=pltpu.CompilerParams(collective_id=0),
    )(x, Wq, K2, V2, Wo)


# device time: 12613 ns/iter; 1.4814x vs baseline; 1.1028x over previous
import jax
import jax.numpy as jnp
from jax import lax
from jax.experimental import pallas as pl
from jax.experimental.pallas import tpu as pltpu

N_DEV = 4
B = 2
SQ = 256
SKV = 256
HALO = 128
SFULL = SKV + 2 * HALO
HQ = 4
DH = 64
HD = HQ * DH
DM = 512
WINDOW = 128
SKV_GLOBAL = N_DEV * SKV


def kernel(x, Wq, K_ext, V_ext, Wo):
    K2 = K_ext.reshape(B, SKV, HD)
    V2 = V_ext.reshape(B, SKV, HD)

    def body(x_ref, wq_ref, k_ref, v_ref, wo_ref, out_ref,
             k_full, v_full, send_sems, recv_sems):
        my = lax.axis_index("i")
        left = (my + N_DEV - 1) % N_DEV
        right = (my + 1) % N_DEV

        barrier_sem = pltpu.get_barrier_semaphore()
        for nbr in (left, right):
            pl.semaphore_signal(
                barrier_sem, inc=1,
                device_id=(nbr,), device_id_type=pl.DeviceIdType.MESH,
            )
        pl.semaphore_wait(barrier_sem, 2)

        k_full[:, HALO:HALO + SKV, :] = k_ref[...].astype(jnp.bfloat16)
        v_full[:, HALO:HALO + SKV, :] = v_ref[...].astype(jnp.bfloat16)

        def halo_rdma(buf, src_off, dst_off, dev, sem_i):
            return pltpu.make_async_remote_copy(
                src_ref=buf.at[:, pl.ds(src_off, HALO), :],
                dst_ref=buf.at[:, pl.ds(dst_off, HALO), :],
                send_sem=send_sems.at[sem_i],
                recv_sem=recv_sems.at[sem_i],
                device_id=(dev,),
                device_id_type=pl.DeviceIdType.MESH,
            )

        rdmas = [
            halo_rdma(k_full, HALO, SKV + HALO, left, 0),
            halo_rdma(k_full, SKV, 0, right, 1),
            halo_rdma(v_full, HALO, SKV + HALO, left, 2),
            halo_rdma(v_full, SKV, 0, right, 3),
        ]
        for r in rdmas:
            r.start()

        wq = wq_ref[...].astype(jnp.bfloat16)
        wo = wo_ref[...].astype(jnp.bfloat16)
        x2 = x_ref[...].reshape(B * SQ, DM).astype(jnp.bfloat16)
        q_all = (lax.dot(x2, wq, preferred_element_type=jnp.float32)
                 * 0.125).astype(jnp.bfloat16)

        qi = lax.broadcasted_iota(jnp.int32, (SQ, SFULL), 0)
        kj = lax.broadcasted_iota(jnp.int32, (SQ, SFULL), 1)
        window = jnp.abs(qi - kj + HALO) <= WINDOW
        kg = kj + my * SKV - HALO
        mask = window & (kg >= 0) & (kg < SKV_GLOBAL)
        mbias = jnp.where(mask, 0.0, -1e9).astype(jnp.float32)

        rdmas[0].wait()
        rdmas[1].wait()

        ws, denoms = [], []
        for b in range(B):
            kb = k_full[b]
            for h in range(HQ):
                qh = q_all[b * SQ:(b + 1) * SQ, h * DH:(h + 1) * DH]
                kh = kb[:, h * DH:(h + 1) * DH]
                s = lax.dot_general(
                    qh, kh, (((1,), (1,)), ((), ())),
                    preferred_element_type=jnp.float32) + mbias
                w = jnp.exp(s.astype(jnp.bfloat16))
                ws.append(w)
                denoms.append(jnp.sum(w, axis=-1, keepdims=True,
                                      dtype=jnp.float32))

        rdmas[2].wait()
        rdmas[3].wait()

        ctxs = []
        for b in range(B):
            vb = v_full[b]
            for h in range(HQ):
                i = b * HQ + h
                ctx_h = lax.dot(
                    ws[i], vb[:, h * DH:(h + 1) * DH],
                    preferred_element_type=jnp.float32)
                ctxs.append(ctx_h / denoms[i])
        ctx = jnp.concatenate(
            [jnp.concatenate(ctxs[b * HQ:(b + 1) * HQ], axis=1)
             for b in range(B)], axis=0)
        out = lax.dot(ctx.astype(jnp.bfloat16), wo,
                      preferred_element_type=jnp.float32)
        for b in range(B):
            out_ref[b] = out[b * SQ:(b + 1) * SQ, :]

    return pl.pallas_call(
        body,
        out_shape=jax.ShapeDtypeStruct((B, SQ, DM), jnp.float32),
        in_specs=[pl.BlockSpec(memory_space=pltpu.VMEM)] * 5,
        out_specs=pl.BlockSpec(memory_space=pltpu.VMEM),
        scratch_shapes=[
            pltpu.VMEM((B, SFULL, HD), jnp.bfloat16),
            pltpu.VMEM((B, SFULL, HD), jnp.bfloat16),
            pltpu.SemaphoreType.DMA((4,)),
            pltpu.SemaphoreType.DMA((4,)),
        ],
        compiler_params=pltpu.CompilerParams(collective_id=0),
    )(x, Wq, K2, V2, Wo)
